# baseline (speedup 1.0000x reference)
import functools
import math

import jax
import jax.numpy as jnp
from jax import lax
from jax.experimental import pallas as pl
from jax.experimental.pallas import tpu as pltpu
from jax.experimental.pallas import tpu_sc as plsc

D_MODEL = 64
SCALE = math.sqrt(D_MODEL)
CHUNK = 128
NC, NS, LANES = 2, 16, 16
NW = NC * NS
NBUF = 2


def _emb_body(chunks_per_w, table_hbm, idx_hbm, out_hbm,
              idx_v, hi_v, ib0, ib1, ob0, ob1, sg0, sg1, so0, so1):
    ibuf = (ib0, ib1)
    obuf = (ob0, ob1)
    sg = (sg0, sg1)
    so = (so0, so1)
    wid = lax.axis_index("s") * NC + lax.axis_index("c")
    rows_per_w = chunks_per_w * CHUNK
    base = wid * rows_per_w
    pltpu.sync_copy(idx_hbm.at[pl.ds(base, rows_per_w)], idx_v)

    def shift_body(i, _):
        sl = pl.ds(i * LANES, LANES)
        hi_v[sl] = idx_v[sl] >> 1
        return 0

    lax.fori_loop(0, rows_per_w // LANES, shift_body, 0, unroll=8)

    def gather(c, slot):
        idx_sl = hi_v.at[pl.ds(c * CHUNK, CHUNK)]
        pltpu.async_copy(table_hbm.at[idx_sl], ibuf[slot], sg[slot])

    for b in range(NBUF):
        gather(b, b)

    def step(it, _):
        for slot in range(NBUF):
            c = it * NBUF + slot
            pltpu.make_async_copy(table_hbm.at[pl.ds(0, CHUNK)],
                                  ibuf[slot], sg[slot]).wait()
            @pl.when(c >= NBUF)
            def _():
                pltpu.make_async_copy(obuf[slot],
                                      out_hbm.at[pl.ds(0, CHUNK)],
                                      so[slot]).wait()

            def grp_body(g, _):
                r0 = g * LANES
                pv = (idx_v[pl.ds(c * CHUNK + r0, LANES)] & 1) * D_MODEL
                for l in range(LANES):
                    cb = pv[l]
                    r = r0 + l
                    for j in range(D_MODEL // LANES):
                        obuf[slot][r, pl.ds(j * LANES, LANES)] = (
                            ibuf[slot][r, pl.ds(cb + j * LANES, LANES)]
                            * SCALE)
                return 0

            lax.fori_loop(0, CHUNK // LANES, grp_body, 0)
            pltpu.async_copy(obuf[slot],
                             out_hbm.at[pl.ds(base + c * CHUNK, CHUNK)],
                             so[slot])

            @pl.when(c + NBUF < chunks_per_w)
            def _():
                gather(c + NBUF, slot)

        return 0

    lax.fori_loop(0, chunks_per_w // NBUF, step, 0)
    for b in range(NBUF):
        pltpu.make_async_copy(obuf[b], out_hbm.at[pl.ds(0, CHUNK)],
                              so[b]).wait()


@jax.jit
def _emb_lookup(lut_pairs, idx):
    n_rows = idx.shape[0]
    chunks_per_w = n_rows // (NW * CHUNK)
    mesh = plsc.VectorSubcoreMesh(core_axis_name="c", subcore_axis_name="s")
    k = pl.kernel(
        functools.partial(_emb_body, chunks_per_w),
        mesh=mesh,
        out_type=jax.ShapeDtypeStruct((n_rows, D_MODEL), jnp.float32),
        scratch_types=[
            pltpu.VMEM((chunks_per_w * CHUNK,), jnp.int32),
            pltpu.VMEM((chunks_per_w * CHUNK,), jnp.int32),
            pltpu.VMEM((CHUNK, 2 * D_MODEL), jnp.float32),
            pltpu.VMEM((CHUNK, 2 * D_MODEL), jnp.float32),
            pltpu.VMEM((CHUNK, D_MODEL), jnp.float32),
            pltpu.VMEM((CHUNK, D_MODEL), jnp.float32),
            pltpu.SemaphoreType.DMA, pltpu.SemaphoreType.DMA,
            pltpu.SemaphoreType.DMA, pltpu.SemaphoreType.DMA,
        ],
    )
    return k(lut_pairs, idx)


def kernel(x, lut):
    n_b, n_s = x.shape
    v, d = lut.shape
    lut_pairs = lut.reshape(v // 2, 2 * d)
    idx = x.reshape(-1).astype(jnp.int32)
    out = _emb_lookup(lut_pairs, idx)
    return out.reshape(n_b, n_s, d)


# speedup vs baseline: 1.0763x; 1.0763x over previous
import functools
import math

import jax
import jax.numpy as jnp
from jax import lax
from jax.experimental import pallas as pl
from jax.experimental.pallas import tpu as pltpu
from jax.experimental.pallas import tpu_sc as plsc

D_MODEL = 64
SCALE = math.sqrt(D_MODEL)
CHUNK = 128
NC, NS, LANES = 2, 16, 16
NW = NC * NS


def _emb_body(chunks_per_w, table_hbm, idx_hbm, out_hbm, idx_v, buf, sem):
    wid = lax.axis_index("s") * NC + lax.axis_index("c")
    rows_per_w = chunks_per_w * CHUNK
    base = wid * rows_per_w
    pltpu.sync_copy(idx_hbm.at[pl.ds(base, rows_per_w)], idx_v)

    def chunk_body(c, _):
        idx_sl = idx_v.at[pl.ds(c * CHUNK, CHUNK)]
        pltpu.async_copy(table_hbm.at[idx_sl], buf, sem).wait()

        def row_body(r, _):
            for j in range(D_MODEL // LANES):
                sl = pl.ds(j * LANES, LANES)
                buf[r, sl] = buf[r, sl] * SCALE
            return 0

        lax.fori_loop(0, CHUNK, row_body, 0, unroll=4)
        pltpu.sync_copy(buf, out_hbm.at[pl.ds(base + c * CHUNK, CHUNK)])
        return 0

    lax.fori_loop(0, chunks_per_w, chunk_body, 0)


@jax.jit
def _emb_lookup(lut, idx):
    n_rows = idx.shape[0]
    chunks_per_w = n_rows // (NW * CHUNK)
    mesh = plsc.VectorSubcoreMesh(core_axis_name="c", subcore_axis_name="s")
    k = pl.kernel(
        functools.partial(_emb_body, chunks_per_w),
        mesh=mesh,
        out_type=jax.ShapeDtypeStruct((n_rows, D_MODEL), jnp.float32),
        scratch_types=[
            pltpu.VMEM((chunks_per_w * CHUNK,), jnp.int32),
            pltpu.VMEM((CHUNK, D_MODEL), jnp.float32),
            pltpu.SemaphoreType.DMA,
        ],
        compiler_params=pltpu.CompilerParams(use_tc_tiling_on_sc=False),
    )
    return k(lut, idx)


def kernel(x, lut):
    b, s = x.shape
    idx = x.reshape(-1).astype(jnp.int32)
    out = _emb_lookup(lut, idx)
    return out.reshape(b, s, D_MODEL)


# speedup vs baseline: 1.1009x; 1.0229x over previous
import functools
import math

import jax
import jax.numpy as jnp
from jax import lax
from jax.experimental import pallas as pl
from jax.experimental.pallas import tpu as pltpu
from jax.experimental.pallas import tpu_sc as plsc

D_MODEL = 64
SCALE = math.sqrt(D_MODEL)
CHUNK = 128
NC, NS, LANES = 2, 16, 16
NW = NC * NS


def _emb_body(chunks_per_w, table_hbm, idx_hbm, out_hbm, idx_v, buf, sem):
    wid = lax.axis_index("s") * NC + lax.axis_index("c")
    rows_per_w = chunks_per_w * CHUNK
    base = wid * rows_per_w
    pltpu.sync_copy(idx_hbm.at[pl.ds(base, rows_per_w)], idx_v)

    def chunk_body(c, _):
        for h in range(2):
            idx_sl = idx_v.at[pl.ds((2 * c + h) * CHUNK, CHUNK)]
            pltpu.async_copy(table_hbm.at[idx_sl],
                             buf.at[pl.ds(h * CHUNK, CHUNK)], sem)
        for h in range(2):
            pltpu.make_async_copy(table_hbm.at[pl.ds(0, CHUNK)],
                                  buf.at[pl.ds(h * CHUNK, CHUNK)], sem).wait()

        def row_body(r, _):
            for j in range(D_MODEL // LANES):
                sl = pl.ds(j * LANES, LANES)
                buf[r, sl] = buf[r, sl] * SCALE
            return 0

        lax.fori_loop(0, 2 * CHUNK, row_body, 0, unroll=8)
        pltpu.sync_copy(buf,
                        out_hbm.at[pl.ds(base + 2 * c * CHUNK, 2 * CHUNK)])
        return 0

    lax.fori_loop(0, chunks_per_w // 2, chunk_body, 0)


@jax.jit
def _emb_lookup(lut, idx):
    n_rows = idx.shape[0]
    chunks_per_w = n_rows // (NW * CHUNK)
    mesh = plsc.VectorSubcoreMesh(core_axis_name="c", subcore_axis_name="s")
    k = pl.kernel(
        functools.partial(_emb_body, chunks_per_w),
        mesh=mesh,
        out_type=jax.ShapeDtypeStruct((n_rows, D_MODEL), jnp.float32),
        scratch_types=[
            pltpu.VMEM((chunks_per_w * CHUNK,), jnp.int32),
            pltpu.VMEM((2 * CHUNK, D_MODEL), jnp.float32),
            pltpu.SemaphoreType.DMA,
        ],
        compiler_params=pltpu.CompilerParams(use_tc_tiling_on_sc=False),
    )
    return k(lut, idx)


def kernel(x, lut):
    b, s = x.shape
    idx = x.reshape(-1).astype(jnp.int32)
    out = _emb_lookup(lut, idx)
    return out.reshape(b, s, D_MODEL)


# speedup vs baseline: 1.1371x; 1.0328x over previous
import functools
import math

import jax
import jax.numpy as jnp
from jax import lax
from jax.experimental import pallas as pl
from jax.experimental.pallas import tpu as pltpu
from jax.experimental.pallas import tpu_sc as plsc

D_MODEL = 64
SCALE = math.sqrt(D_MODEL)
CHUNK = 128
NC, NS, LANES = 2, 16, 16
NW = NC * NS


def _emb_body(chunks_per_w, table_hbm, idx_hbm, out_hbm,
              idx_v, b0, b1, s0, s1):
    bufs = (b0, b1)
    sems = (s0, s1)
    n_pairs = chunks_per_w // 2
    wid = lax.axis_index("s") * NC + lax.axis_index("c")
    rows_per_w = chunks_per_w * CHUNK
    base = wid * rows_per_w
    pltpu.sync_copy(idx_hbm.at[pl.ds(base, rows_per_w)], idx_v)

    def fire(p, k):
        for h in range(2):
            idx_sl = idx_v.at[pl.ds((2 * p + h) * CHUNK, CHUNK)]
            pltpu.async_copy(table_hbm.at[idx_sl],
                             bufs[k].at[pl.ds(h * CHUNK, CHUNK)], sems[k])

    def drain(k):
        for h in range(2):
            pltpu.make_async_copy(table_hbm.at[pl.ds(0, CHUNK)],
                                  bufs[k].at[pl.ds(h * CHUNK, CHUNK)],
                                  sems[k]).wait()

    def process(p, k):
        drain(k)

        @pl.when(p + 1 < n_pairs)
        def _():
            fire(p + 1, 1 - k)

        def row_body(r, _):
            for j in range(D_MODEL // LANES):
                sl = pl.ds(j * LANES, LANES)
                bufs[k][r, sl] = bufs[k][r, sl] * SCALE
            return 0

        lax.fori_loop(0, 2 * CHUNK, row_body, 0, unroll=8)
        pltpu.sync_copy(bufs[k],
                        out_hbm.at[pl.ds(base + 2 * p * CHUNK, 2 * CHUNK)])

    fire(0, 0)

    def step(i, _):
        for k in range(2):
            process(i * 2 + k, k)
        return 0

    lax.fori_loop(0, n_pairs // 2, step, 0)
    if n_pairs % 2:
        process(n_pairs - 1, 0)


@jax.jit
def _emb_lookup(lut, idx):
    n_rows = idx.shape[0]
    chunks_per_w = n_rows // (NW * CHUNK)
    mesh = plsc.VectorSubcoreMesh(core_axis_name="c", subcore_axis_name="s")
    k = pl.kernel(
        functools.partial(_emb_body, chunks_per_w),
        mesh=mesh,
        out_type=jax.ShapeDtypeStruct((n_rows, D_MODEL), jnp.float32),
        scratch_types=[
            pltpu.VMEM((chunks_per_w * CHUNK,), jnp.int32),
            pltpu.VMEM((2 * CHUNK, D_MODEL), jnp.float32),
            pltpu.VMEM((2 * CHUNK, D_MODEL), jnp.float32),
            pltpu.SemaphoreType.DMA, pltpu.SemaphoreType.DMA,
        ],
        compiler_params=pltpu.CompilerParams(use_tc_tiling_on_sc=False),
    )
    return k(lut, idx)


def kernel(x, lut):
    b, s = x.shape
    idx = x.reshape(-1).astype(jnp.int32)
    out = _emb_lookup(lut, idx)
    return out.reshape(b, s, D_MODEL)
